# baseline (speedup 1.0000x reference)
import jax
import jax.numpy as jnp
from jax.experimental import pallas as pl
from jax.experimental.pallas import tpu as pltpu

_N_PAPER = 100000
_N_AUTHOR = 50000
_N_FIELD = 10000
_EMBED = 128
_CHUNK = 10000
_PB = _N_PAPER // _CHUNK
_AB = _N_AUTHOR // _CHUNK
_FB = _N_FIELD // _CHUNK


def _concat_kernel(p_ref, a_ref, f_ref, o_ref, sem):
    i = pl.program_id(0)

    @pl.when(i < _PB)
    def _():
        pltpu.make_async_copy(
            p_ref.at[pl.ds(i * _CHUNK, _CHUNK)], o_ref, sem).start()

    @pl.when(jnp.logical_and(i >= _PB, i < _PB + _AB))
    def _():
        pltpu.make_async_copy(
            a_ref.at[pl.ds((i - _PB) * _CHUNK, _CHUNK)], o_ref, sem).start()

    @pl.when(i >= _PB + _AB)
    def _():
        pltpu.make_async_copy(
            f_ref.at[pl.ds((i - _PB - _AB) * _CHUNK, _CHUNK)], o_ref,
            sem).start()

    pltpu.make_async_copy(p_ref.at[pl.ds(0, _CHUNK)], o_ref, sem).wait()


def kernel(embed_paper, embed_author, embed_field):
    total = _N_PAPER + _N_AUTHOR + _N_FIELD
    return pl.pallas_call(
        _concat_kernel,
        grid=(total // _CHUNK,),
        out_shape=jax.ShapeDtypeStruct((total, _EMBED), jnp.float32),
        in_specs=[
            pl.BlockSpec(memory_space=pl.ANY),
            pl.BlockSpec(memory_space=pl.ANY),
            pl.BlockSpec(memory_space=pl.ANY),
        ],
        out_specs=pl.BlockSpec((_CHUNK, _EMBED), lambda i: (i, 0)),
        scratch_shapes=[pltpu.SemaphoreType.DMA],
    )(embed_paper, embed_author, embed_field)


# speedup vs baseline: 1.3378x; 1.3378x over previous
import jax
import jax.numpy as jnp
from jax.experimental import pallas as pl
from jax.experimental.pallas import tpu as pltpu

_N_PAPER = 100000
_N_AUTHOR = 50000
_N_FIELD = 10000
_EMBED = 128
_CHUNK = 10000
_PB = _N_PAPER // _CHUNK
_AB = _N_AUTHOR // _CHUNK
_FB = _N_FIELD // _CHUNK


def _concat_kernel(p_ref, a_ref, f_ref, o_ref):
    i = pl.program_id(0)

    @pl.when(i < _PB)
    def _():
        o_ref[...] = p_ref[...]

    @pl.when(jnp.logical_and(i >= _PB, i < _PB + _AB))
    def _():
        o_ref[...] = a_ref[...]

    @pl.when(i >= _PB + _AB)
    def _():
        o_ref[...] = f_ref[...]


def kernel(embed_paper, embed_author, embed_field):
    total = _N_PAPER + _N_AUTHOR + _N_FIELD
    return pl.pallas_call(
        _concat_kernel,
        grid=(_PB + _AB + _FB,),
        out_shape=jax.ShapeDtypeStruct((total, _EMBED), jnp.float32),
        in_specs=[
            pl.BlockSpec((_CHUNK, _EMBED),
                         lambda i: (jnp.minimum(i, _PB - 1), 0)),
            pl.BlockSpec((_CHUNK, _EMBED),
                         lambda i: (jnp.clip(i - _PB, 0, _AB - 1), 0)),
            pl.BlockSpec((_CHUNK, _EMBED),
                         lambda i: (jnp.clip(i - _PB - _AB, 0, _FB - 1), 0)),
        ],
        out_specs=pl.BlockSpec((_CHUNK, _EMBED), lambda i: (i, 0)),
        compiler_params=pltpu.CompilerParams(
            dimension_semantics=("parallel",)),
    )(embed_paper, embed_author, embed_field)
